# baseline (speedup 1.0000x reference)
import functools

import jax
import jax.numpy as jnp
from jax import lax
from jax.experimental import pallas as pl
from jax.experimental.pallas import tpu as pltpu
from jax.experimental.pallas import tpu_sc as plsc

_NC = 2
_NS = 16
_NW = _NC * _NS
_CHUNK = 4096
_GROUPS = _CHUNK // 16


def _vgather(tab, idx):
    return lax.gather(
        tab, idx[:, None],
        lax.GatherDimensionNumbers(
            offset_dims=(), collapsed_slice_dims=(0,), start_index_map=(0,)),
        (1,), mode=lax.GatherScatterMode.PROMISE_IN_BOUNDS)


def _rsqrt2(s):
    i = lax.bitcast_convert_type(s, jnp.int32)
    y = lax.bitcast_convert_type(jnp.int32(0x5F3759DF) - (i >> 1), jnp.float32)
    hs = 0.5 * s
    y = y * (1.5 - hs * y * y)
    y = y * (1.5 - hs * y * y)
    return y


def _sc_body(rows_per_worker, qh, ch, sh, out_h, inb, outb, ctab_v, stab_v):
    iters = rows_per_worker // _CHUNK
    wid = lax.axis_index("c") * _NS + lax.axis_index("s")
    base_w = wid * (rows_per_worker * 4)

    pltpu.sync_copy(ch, ctab_v)
    pltpu.sync_copy(sh, stab_v)
    ctv = ctab_v[...]
    stv = stab_v[...]
    iota4 = lax.iota(jnp.int32, 16) * 4

    def chunk_body(g, carry):
        off = base_w + g * (_CHUNK * 4)
        pltpu.sync_copy(qh.at[pl.ds(off, _CHUNK * 4)], inb)

        def grp(j, c2):
            iw = iota4 + j * 64
            w = plsc.load_gather(inb, [iw])
            x = plsc.load_gather(inb, [iw + 1])
            y = plsc.load_gather(inb, [iw + 2])
            z = plsc.load_gather(inb, [iw + 3])

            ww = w * w
            sv = x * x + y * y + z * z
            s = sv + ww
            norms = jnp.maximum(s * _rsqrt2(s), 1e-6)

            lo = jnp.zeros((16,), jnp.int32)
            for sz in (8, 4, 2, 1):
                cm = _vgather(ctv, lo + sz)
                lo = jnp.where(w < cm * norms, lo + sz, lo)

            cq = _vgather(ctv, lo)
            sq = _vgather(stv, lo)
            t = sq * (norms * _rsqrt2(sv))

            plsc.store_scatter(outb, [iw], cq * norms)
            plsc.store_scatter(outb, [iw + 1], x * t)
            plsc.store_scatter(outb, [iw + 2], y * t)
            plsc.store_scatter(outb, [iw + 3], z * t)
            return c2

        lax.fori_loop(0, _GROUPS, grp, 0)
        pltpu.sync_copy(outb, out_h.at[pl.ds(off, _CHUNK * 4)])
        return carry

    lax.fori_loop(0, iters, chunk_body, 0)


@functools.partial(jax.jit, static_argnums=(3,))
def _run_sc(qf, ctab, stab, n_rows):
    rows_per_worker = n_rows // _NW
    mesh = plsc.VectorSubcoreMesh(
        core_axis_name="c", subcore_axis_name="s",
        num_cores=_NC, num_subcores=_NS)
    f = pl.kernel(
        functools.partial(_sc_body, rows_per_worker),
        out_type=jax.ShapeDtypeStruct((n_rows * 4,), jnp.float32),
        mesh=mesh,
        scratch_types=[
            pltpu.VMEM((_CHUNK * 4,), jnp.float32),
            pltpu.VMEM((_CHUNK * 4,), jnp.float32),
            pltpu.VMEM((16,), jnp.float32),
            pltpu.VMEM((16,), jnp.float32),
        ],
        compiler_params=pltpu.CompilerParams(needs_layout_passes=False),
    )
    return f(qf, ctab, stab)


def kernel(q_weights, bins):
    n_rows = q_weights.shape[0]
    half = bins * 0.5
    ctab = jnp.cos(half)
    stab = jnp.sin(half)
    outf = _run_sc(q_weights.reshape(-1), ctab, stab, n_rows)
    return outf.reshape(q_weights.shape)
